# Initial kernel scaffold; baseline (speedup 1.0000x reference)
import jax
import jax.numpy as jnp
from jax.experimental import pallas as pl


def kernel(inputs, edge_index, batch_indexes, W1, b1, W2, b2, W3, b3, Wlin, blin):
    raise NotImplementedError("write your pallas kernel here")


# speedup vs baseline: 12.9265x; 12.9265x over previous
import functools

import jax
import jax.numpy as jnp
from jax import lax
from jax.experimental import pallas as pl
from jax.experimental.pallas import tpu as pltpu
from jax.experimental.pallas import tpu_sc as plsc

NC = 2
NS = 16
CH = 80
G = 64
F32 = jnp.float32


def _sc_mesh():
  return plsc.VectorSubcoreMesh(core_axis_name="c", subcore_axis_name="s")


def _row_split(N):
  r_sm = 8 * (N // (8 * NS))
  r_lg = N - r_sm * (NS - 1)
  return r_sm, r_lg


def _for_tile_rows(s, r_sm, r_lg, do):
  @pl.when(s < NS - 1)
  def _():
    do(pl.multiple_of(s * r_sm, 8), r_sm)

  @pl.when(s == NS - 1)
  def _():
    do((NS - 1) * r_sm, r_lg)


def _make_sc_edge_layer(N, H, E):
  NW = NC * NS
  epw = E // NW
  nchunk = epw // CH
  r_sm, r_lg = _row_split(N)

  @functools.partial(
      pl.kernel,
      out_type=jax.ShapeDtypeStruct((NC, N, H), F32),
      mesh=_sc_mesh(),
      compiler_params=pltpu.CompilerParams(use_tc_tiling_on_sc=False),
      scratch_types=[
          pltpu.VMEM((CH,), jnp.int32),
          pltpu.VMEM((CH,), jnp.int32),
          pltpu.VMEM((CH, H), F32),
          pltpu.VMEM_SHARED((N, H), F32),
      ],
  )
  def k(y_hbm, src_hbm, dst_hbm, zeros_hbm, out_hbm, idx_s, idx_d, rows, acc):
    c = lax.axis_index("c")
    s = lax.axis_index("s")
    base = (c * NS + s) * epw
    _for_tile_rows(s, r_sm, r_lg, lambda o, n: pltpu.sync_copy(
        zeros_hbm.at[pl.ds(0, n)], acc.at[pl.ds(o, n)]))
    plsc.subcore_barrier()

    def body(j, carry):
      off = pl.multiple_of(base + j * CH, 8)
      pltpu.sync_copy(src_hbm.at[pl.ds(off, CH)], idx_s)
      pltpu.sync_copy(dst_hbm.at[pl.ds(off, CH)], idx_d)
      pltpu.sync_copy(y_hbm.at[idx_s], rows)
      pltpu.sync_copy(rows, acc.at[idx_d], add=True)
      return carry

    lax.fori_loop(0, nchunk, body, 0)
    plsc.subcore_barrier()
    _for_tile_rows(s, r_sm, r_lg, lambda o, n: pltpu.sync_copy(
        acc.at[pl.ds(o, n)], out_hbm.at[c].at[pl.ds(o, n)]))

  return k


def _make_sc_deg(N, E, W):
  NW = NC * NS
  epw = E // NW
  nchunk = epw // CH
  r_sm, r_lg = _row_split(N)

  @functools.partial(
      pl.kernel,
      out_type=jax.ShapeDtypeStruct((NC, N, W), F32),
      mesh=_sc_mesh(),
      compiler_params=pltpu.CompilerParams(use_tc_tiling_on_sc=False),
      scratch_types=[
          pltpu.VMEM((CH,), jnp.int32),
          pltpu.VMEM((CH, W), F32),
          pltpu.VMEM_SHARED((N, W), F32),
      ],
  )
  def k(dst_hbm, ones_hbm, zeros_hbm, out_hbm, idx_d, ones_v, acc):
    c = lax.axis_index("c")
    s = lax.axis_index("s")
    base = (c * NS + s) * epw
    _for_tile_rows(s, r_sm, r_lg, lambda o, n: pltpu.sync_copy(
        zeros_hbm.at[pl.ds(0, n)], acc.at[pl.ds(o, n)]))
    pltpu.sync_copy(ones_hbm, ones_v)
    plsc.subcore_barrier()

    def body(j, carry):
      off = pl.multiple_of(base + j * CH, 8)
      pltpu.sync_copy(dst_hbm.at[pl.ds(off, CH)], idx_d)
      pltpu.sync_copy(ones_v, acc.at[idx_d], add=True)
      return carry

    lax.fori_loop(0, nchunk, body, 0)
    plsc.subcore_barrier()
    _for_tile_rows(s, r_sm, r_lg, lambda o, n: pltpu.sync_copy(
        acc.at[pl.ds(o, n)], out_hbm.at[c].at[pl.ds(o, n)]))

  return k


def _tc_first(x, W1, degp):
  N = x.shape[0]
  H = W1.shape[1]

  def body(x_ref, w_ref, degp_ref, dinv_ref, y_ref):
    deg = degp_ref[0, :, 0:1] + degp_ref[1, :, 0:1] + 1.0
    dinv = lax.rsqrt(deg)
    dinv_ref[...] = dinv
    xw = jnp.dot(x_ref[...], w_ref[...], preferred_element_type=F32)
    y_ref[...] = xw * dinv

  return pl.pallas_call(
      body,
      out_shape=(jax.ShapeDtypeStruct((N, 1), F32),
                 jax.ShapeDtypeStruct((N, H), F32)),
  )(x, W1, degp)


def _tc_mid(p, y, dinv, b, Wn):
  N, H = y.shape

  def body(p_ref, y_ref, dinv_ref, b_ref, w_ref, out_ref):
    acc = p_ref[0] + p_ref[1] + y_ref[...]
    x = acc * dinv_ref[...] + b_ref[...]
    out_ref[...] = jnp.dot(x, w_ref[...], preferred_element_type=F32) * dinv_ref[...]

  return pl.pallas_call(
      body, out_shape=jax.ShapeDtypeStruct((N, H), F32),
  )(p, y, dinv, b, Wn)


def _tc_last(p, y, dinv, b, seg, Wlin, blin):
  N, H = y.shape
  C = Wlin.shape[1]

  def body(p_ref, y_ref, dinv_ref, b_ref, seg_ref, wl_ref, bl_ref, out_ref):
    x3 = (p_ref[0] + p_ref[1] + y_ref[...]) * dinv_ref[...] + b_ref[...]
    gids = lax.broadcasted_iota(jnp.int32, (G, N), 0)
    M = (seg_ref[...] == gids).astype(F32)
    sums = jnp.dot(M, x3, preferred_element_type=F32)
    counts = jnp.sum(M, axis=1, keepdims=True)
    pooled = sums / jnp.maximum(counts, 1.0)
    out_ref[...] = jnp.dot(pooled, wl_ref[...], preferred_element_type=F32) + bl_ref[...]

  return pl.pallas_call(
      body, out_shape=jax.ShapeDtypeStruct((G, C), F32),
  )(p, y, dinv, b, seg, Wlin, blin)


def kernel(inputs, edge_index, batch_indexes, W1, b1, W2, b2, W3, b3, Wlin, blin):
  N, D = inputs.shape
  H = W1.shape[1]
  E = edge_index.shape[1]
  C = Wlin.shape[1]
  src = edge_index[0]
  dst = edge_index[1]

  DW = 16
  _, r_lg = _row_split(N)
  zeros_h = jnp.zeros((r_lg, H), F32)
  zeros_d = jnp.zeros((r_lg, DW), F32)
  ones_d = jnp.ones((CH, DW), F32)

  sc_deg = _make_sc_deg(N, E, DW)
  sc_layer = _make_sc_edge_layer(N, H, E)

  degp = sc_deg(dst, ones_d, zeros_d)
  dinv, y1 = _tc_first(inputs, W1, degp)
  p1 = sc_layer(y1, src, dst, zeros_h)
  y2 = _tc_mid(p1, y1, dinv, b1.reshape(1, H), W2)
  p2 = sc_layer(y2, src, dst, zeros_h)
  y3 = _tc_mid(p2, y2, dinv, b2.reshape(1, H), W3)
  p3 = sc_layer(y3, src, dst, zeros_h)
  seg = batch_indexes.reshape(1, N)
  return _tc_last(p3, y3, dinv, b3.reshape(1, H), seg, Wlin, blin.reshape(1, C))


# speedup vs baseline: 40.7249x; 3.1505x over previous
import functools

import jax
import jax.numpy as jnp
from jax import lax
from jax.experimental import pallas as pl
from jax.experimental.pallas import tpu as pltpu
from jax.experimental.pallas import tpu_sc as plsc

NC = 2
NS = 16
CH = 80
NB = 5
G = 64
F32 = jnp.float32


def _sc_mesh():
  return plsc.VectorSubcoreMesh(core_axis_name="c", subcore_axis_name="s")


def _row_split(N):
  r_sm = 8 * (N // (8 * NS))
  r_lg = N - r_sm * (NS - 1)
  return r_sm, r_lg


def _for_tile_rows(s, r_sm, r_lg, do):
  @pl.when(s < NS - 1)
  def _():
    do(pl.multiple_of(s * r_sm, 8), r_sm)

  @pl.when(s == NS - 1)
  def _():
    do((NS - 1) * r_sm, r_lg)


def _make_sc_edge_layer(N, H, E):
  NW = NC * NS
  epw = E // NW
  nchunk = epw // CH
  nouter = nchunk // NB
  r_sm, r_lg = _row_split(N)

  @functools.partial(
      pl.kernel,
      out_type=jax.ShapeDtypeStruct((NC, N, H), F32),
      mesh=_sc_mesh(),
      compiler_params=pltpu.CompilerParams(use_tc_tiling_on_sc=False),
      scratch_types=[
          pltpu.VMEM((nchunk, CH), jnp.int32),
          pltpu.VMEM((nchunk, CH), jnp.int32),
          tuple(pltpu.VMEM((CH, H), F32) for _ in range(NB)),
          tuple(pltpu.SemaphoreType.DMA for _ in range(NB)),
          tuple(pltpu.SemaphoreType.DMA for _ in range(NB)),
          pltpu.VMEM_SHARED((N, H), F32),
      ],
  )
  def k(y_hbm, src3, dst3, zeros_hbm, out_hbm, srcv, dstv, rows, gsem, ssem, acc):
    c = lax.axis_index("c")
    s = lax.axis_index("s")
    wid = c * NS + s
    pltpu.sync_copy(src3.at[wid], srcv)
    pltpu.sync_copy(dst3.at[wid], dstv)
    _for_tile_rows(s, r_sm, r_lg, lambda o, n: pltpu.sync_copy(
        zeros_hbm.at[pl.ds(0, n)], acc.at[pl.ds(o, n)]))
    plsc.subcore_barrier()

    def g_start(j, b):
      pltpu.async_copy(y_hbm.at[srcv.at[j]], rows[b], gsem[b])

    def g_wait(j, b):
      pltpu.make_async_copy(y_hbm.at[srcv.at[j]], rows[b], gsem[b]).wait()

    def s_start(j, b):
      pltpu.async_copy(rows[b], acc.at[dstv.at[j]], ssem[b], add=True)

    def s_wait(j, b):
      pltpu.make_async_copy(rows[b], acc.at[dstv.at[j]], ssem[b]).wait()

    for b in range(NB - 1):
      g_start(b, b)

    def outer(i, carry):
      for b in range(NB):
        j = i * NB + b
        g_wait(j, b)
        s_start(j, b)
        b2 = (b + NB - 1) % NB
        jn = j + NB - 1
        if b == 0:

          @pl.when(i > 0)
          def _():
            s_wait(j - 1, b2)
            g_start(jn, b2)

          @pl.when(i == 0)
          def _():
            g_start(jn, b2)
        else:

          @pl.when(jn < nchunk)
          def _():
            s_wait(j - 1, b2)
            g_start(jn, b2)
      return carry

    lax.fori_loop(0, nouter, outer, 0)
    for b in range(NB):
      s_wait(nchunk - NB + b, b)
    plsc.subcore_barrier()
    _for_tile_rows(s, r_sm, r_lg, lambda o, n: pltpu.sync_copy(
        acc.at[pl.ds(o, n)], out_hbm.at[c].at[pl.ds(o, n)]))

  return k


def _make_sc_deg(N, E, W):
  NW = NC * NS
  epw = E // NW
  nchunk = epw // CH
  r_sm, r_lg = _row_split(N)

  @functools.partial(
      pl.kernel,
      out_type=jax.ShapeDtypeStruct((NC, N, W), F32),
      mesh=_sc_mesh(),
      compiler_params=pltpu.CompilerParams(use_tc_tiling_on_sc=False),
      scratch_types=[
          pltpu.VMEM((nchunk, CH), jnp.int32),
          pltpu.VMEM((CH, W), F32),
          tuple(pltpu.SemaphoreType.DMA for _ in range(NB)),
          pltpu.VMEM_SHARED((N, W), F32),
      ],
  )
  def k(dst3, ones_hbm, zeros_hbm, out_hbm, dstv, ones_v, ssem, acc):
    c = lax.axis_index("c")
    s = lax.axis_index("s")
    wid = c * NS + s
    pltpu.sync_copy(dst3.at[wid], dstv)
    _for_tile_rows(s, r_sm, r_lg, lambda o, n: pltpu.sync_copy(
        zeros_hbm.at[pl.ds(0, n)], acc.at[pl.ds(o, n)]))
    pltpu.sync_copy(ones_hbm, ones_v)
    plsc.subcore_barrier()

    def s_start(j, b):
      pltpu.async_copy(ones_v, acc.at[dstv.at[j]], ssem[b], add=True)

    def s_wait(j, b):
      pltpu.make_async_copy(ones_v, acc.at[dstv.at[j]], ssem[b]).wait()

    def outer(i, carry):
      for b in range(NB):
        j = i * NB + b

        @pl.when(i > 0)
        def _():
          s_wait(j - NB, b)

        s_start(j, b)
      return carry

    lax.fori_loop(0, nchunk // NB, outer, 0)
    for b in range(NB):
      s_wait(nchunk - NB + b, b)
    plsc.subcore_barrier()
    _for_tile_rows(s, r_sm, r_lg, lambda o, n: pltpu.sync_copy(
        acc.at[pl.ds(o, n)], out_hbm.at[c].at[pl.ds(o, n)]))

  return k


def _tc_first(x, W1, degp):
  N = x.shape[0]
  H = W1.shape[1]

  def body(x_ref, w_ref, degp_ref, dinv_ref, y_ref):
    deg = degp_ref[0, :, 0:1] + degp_ref[1, :, 0:1] + 1.0
    dinv = lax.rsqrt(deg)
    dinv_ref[...] = dinv
    xw = jnp.dot(x_ref[...], w_ref[...], preferred_element_type=F32)
    y_ref[...] = xw * dinv

  return pl.pallas_call(
      body,
      out_shape=(jax.ShapeDtypeStruct((N, 1), F32),
                 jax.ShapeDtypeStruct((N, H), F32)),
  )(x, W1, degp)


def _tc_mid(p, y, dinv, b, Wn):
  N, H = y.shape

  def body(p_ref, y_ref, dinv_ref, b_ref, w_ref, out_ref):
    acc = p_ref[0] + p_ref[1] + y_ref[...]
    x = acc * dinv_ref[...] + b_ref[...]
    out_ref[...] = jnp.dot(x, w_ref[...], preferred_element_type=F32) * dinv_ref[...]

  return pl.pallas_call(
      body, out_shape=jax.ShapeDtypeStruct((N, H), F32),
  )(p, y, dinv, b, Wn)


def _tc_last(p, y, dinv, b, seg, Wlin, blin):
  N, H = y.shape
  C = Wlin.shape[1]

  def body(p_ref, y_ref, dinv_ref, b_ref, seg_ref, wl_ref, bl_ref, out_ref):
    x3 = (p_ref[0] + p_ref[1] + y_ref[...]) * dinv_ref[...] + b_ref[...]
    gids = lax.broadcasted_iota(jnp.int32, (G, N), 0)
    M = (seg_ref[...] == gids).astype(F32)
    sums = jnp.dot(M, x3, preferred_element_type=F32)
    counts = jnp.sum(M, axis=1, keepdims=True)
    pooled = sums / jnp.maximum(counts, 1.0)
    out_ref[...] = jnp.dot(pooled, wl_ref[...], preferred_element_type=F32) + bl_ref[...]

  return pl.pallas_call(
      body, out_shape=jax.ShapeDtypeStruct((G, C), F32),
  )(p, y, dinv, b, seg, Wlin, blin)


def kernel(inputs, edge_index, batch_indexes, W1, b1, W2, b2, W3, b3, Wlin, blin):
  N, D = inputs.shape
  H = W1.shape[1]
  E = edge_index.shape[1]
  C = Wlin.shape[1]
  NW = NC * NS
  nchunk = E // (NW * CH)
  src3 = edge_index[0].reshape(NW, nchunk, CH)
  dst3 = edge_index[1].reshape(NW, nchunk, CH)

  DW = 16
  _, r_lg = _row_split(N)
  zeros_h = jnp.zeros((r_lg, H), F32)
  zeros_d = jnp.zeros((r_lg, DW), F32)
  ones_d = jnp.ones((CH, DW), F32)

  sc_deg = _make_sc_deg(N, E, DW)
  sc_layer = _make_sc_edge_layer(N, H, E)

  degp = sc_deg(dst3, ones_d, zeros_d)
  dinv, y1 = _tc_first(inputs, W1, degp)
  p1 = sc_layer(y1, src3, dst3, zeros_h)
  y2 = _tc_mid(p1, y1, dinv, b1.reshape(1, H), W2)
  p2 = sc_layer(y2, src3, dst3, zeros_h)
  y3 = _tc_mid(p2, y2, dinv, b2.reshape(1, H), W3)
  p3 = sc_layer(y3, src3, dst3, zeros_h)
  seg = batch_indexes.reshape(1, N)
  return _tc_last(p3, y3, dinv, b3.reshape(1, H), seg, Wlin, blin.reshape(1, C))


# speedup vs baseline: 42.6446x; 1.0471x over previous
import functools

import jax
import jax.numpy as jnp
from jax import lax
from jax.experimental import pallas as pl
from jax.experimental.pallas import tpu as pltpu
from jax.experimental.pallas import tpu_sc as plsc

NC = 2
NS = 16
CH = 128
NB = 6
G = 64
F32 = jnp.float32


def _sc_mesh():
  return plsc.VectorSubcoreMesh(core_axis_name="c", subcore_axis_name="s")


def _row_split(N):
  r_sm = 8 * (N // (8 * NS))
  r_lg = N - r_sm * (NS - 1)
  return r_sm, r_lg


def _for_tile_rows(s, r_sm, r_lg, do):
  @pl.when(s < NS - 1)
  def _():
    do(pl.multiple_of(s * r_sm, 8), r_sm)

  @pl.when(s == NS - 1)
  def _():
    do((NS - 1) * r_sm, r_lg)


def _make_sc_edge_layer(N, H, E):
  NW = NC * NS
  epw = E // NW
  nchunk = epw // CH
  tail = epw - nchunk * CH
  nouter = nchunk // NB
  r_sm, r_lg = _row_split(N)

  @functools.partial(
      pl.kernel,
      out_type=jax.ShapeDtypeStruct((NC, N, H), F32),
      mesh=_sc_mesh(),
      compiler_params=pltpu.CompilerParams(use_tc_tiling_on_sc=False),
      scratch_types=[
          pltpu.VMEM((nchunk, CH), jnp.int32),
          pltpu.VMEM((nchunk, CH), jnp.int32),
          pltpu.VMEM((tail,), jnp.int32),
          pltpu.VMEM((tail,), jnp.int32),
          pltpu.VMEM((tail, H), F32),
          tuple(pltpu.VMEM((CH, H), F32) for _ in range(NB)),
          tuple(pltpu.SemaphoreType.DMA for _ in range(NB)),
          tuple(pltpu.SemaphoreType.DMA for _ in range(NB)),
          pltpu.VMEM_SHARED((N, H), F32),
      ],
  )
  def k(y_hbm, srcm, dstm, srct, dstt, zeros_hbm, out_hbm,
        srcv, dstv, srctv, dsttv, rowst, rows, gsem, ssem, acc):
    c = lax.axis_index("c")
    s = lax.axis_index("s")
    wid = c * NS + s
    i0 = pltpu.async_copy(srcm.at[wid], srcv, gsem[0])
    i1 = pltpu.async_copy(dstm.at[wid], dstv, gsem[1])
    i2 = pltpu.async_copy(srct.at[wid], srctv, gsem[2])
    i3 = pltpu.async_copy(dstt.at[wid], dsttv, gsem[3])
    _for_tile_rows(s, r_sm, r_lg, lambda o, n: pltpu.sync_copy(
        zeros_hbm.at[pl.ds(0, n)], acc.at[pl.ds(o, n)]))
    i0.wait()
    i1.wait()
    i2.wait()
    i3.wait()
    plsc.subcore_barrier()
    pltpu.sync_copy(y_hbm.at[srctv], rowst)
    pltpu.sync_copy(rowst, acc.at[dsttv], add=True)

    def g_start(j, b):
      pltpu.async_copy(y_hbm.at[srcv.at[j]], rows[b], gsem[b])

    def g_wait(j, b):
      pltpu.make_async_copy(y_hbm.at[srcv.at[j]], rows[b], gsem[b]).wait()

    def s_start(j, b):
      pltpu.async_copy(rows[b], acc.at[dstv.at[j]], ssem[b], add=True)

    def s_wait(j, b):
      pltpu.make_async_copy(rows[b], acc.at[dstv.at[j]], ssem[b]).wait()

    for b in range(NB - 1):
      g_start(b, b)

    def outer(i, carry):
      for b in range(NB):
        j = i * NB + b
        g_wait(j, b)
        s_start(j, b)
        b2 = (b + NB - 1) % NB
        jn = j + NB - 1
        if b == 0:

          @pl.when(i > 0)
          def _():
            s_wait(j - 1, b2)
            g_start(jn, b2)

          @pl.when(i == 0)
          def _():
            g_start(jn, b2)
        else:

          @pl.when(jn < nchunk)
          def _():
            s_wait(j - 1, b2)
            g_start(jn, b2)
      return carry

    lax.fori_loop(0, nouter, outer, 0)
    for b in range(NB):
      s_wait(nchunk - NB + b, b)
    plsc.subcore_barrier()
    _for_tile_rows(s, r_sm, r_lg, lambda o, n: pltpu.sync_copy(
        acc.at[pl.ds(o, n)], out_hbm.at[c].at[pl.ds(o, n)]))

  return k


def _make_sc_deg(N, E, W):
  NW = NC * NS
  epw = E // NW
  nchunk = epw // CH
  tail = epw - nchunk * CH
  r_sm, r_lg = _row_split(N)

  @functools.partial(
      pl.kernel,
      out_type=jax.ShapeDtypeStruct((NC, N, W), F32),
      mesh=_sc_mesh(),
      compiler_params=pltpu.CompilerParams(use_tc_tiling_on_sc=False),
      scratch_types=[
          pltpu.VMEM((nchunk, CH), jnp.int32),
          pltpu.VMEM((tail,), jnp.int32),
          pltpu.VMEM((CH, W), F32),
          pltpu.VMEM((tail, W), F32),
          tuple(pltpu.SemaphoreType.DMA for _ in range(NB)),
          pltpu.VMEM_SHARED((N, W), F32),
      ],
  )
  def k(dstm, dstt, ones_hbm, zeros_hbm, out_hbm, dstv, dsttv, ones_v, onest_v,
        ssem, acc):
    c = lax.axis_index("c")
    s = lax.axis_index("s")
    wid = c * NS + s
    i0 = pltpu.async_copy(dstm.at[wid], dstv, ssem[0])
    i1 = pltpu.async_copy(dstt.at[wid], dsttv, ssem[1])
    i2 = pltpu.async_copy(ones_hbm, ones_v, ssem[2])
    i3 = pltpu.async_copy(ones_hbm.at[pl.ds(0, tail)], onest_v, ssem[3])
    _for_tile_rows(s, r_sm, r_lg, lambda o, n: pltpu.sync_copy(
        zeros_hbm.at[pl.ds(0, n)], acc.at[pl.ds(o, n)]))
    i0.wait()
    i1.wait()
    i2.wait()
    i3.wait()
    plsc.subcore_barrier()
    pltpu.sync_copy(onest_v, acc.at[dsttv], add=True)

    def s_start(j, b):
      pltpu.async_copy(ones_v, acc.at[dstv.at[j]], ssem[b], add=True)

    def s_wait(j, b):
      pltpu.make_async_copy(ones_v, acc.at[dstv.at[j]], ssem[b]).wait()

    def outer(i, carry):
      for b in range(NB):
        j = i * NB + b

        @pl.when(i > 0)
        def _():
          s_wait(j - NB, b)

        s_start(j, b)
      return carry

    lax.fori_loop(0, nchunk // NB, outer, 0)
    for b in range(NB):
      s_wait(nchunk - NB + b, b)
    plsc.subcore_barrier()
    _for_tile_rows(s, r_sm, r_lg, lambda o, n: pltpu.sync_copy(
        acc.at[pl.ds(o, n)], out_hbm.at[c].at[pl.ds(o, n)]))

  return k


def _tc_first(x, W1, degp):
  N = x.shape[0]
  H = W1.shape[1]

  def body(x_ref, w_ref, degp_ref, dinv_ref, y_ref):
    deg = degp_ref[0, :, 0:1] + degp_ref[1, :, 0:1] + 1.0
    dinv = lax.rsqrt(deg)
    dinv_ref[...] = dinv
    xw = jnp.dot(x_ref[...], w_ref[...], preferred_element_type=F32)
    y_ref[...] = xw * dinv

  return pl.pallas_call(
      body,
      out_shape=(jax.ShapeDtypeStruct((N, 1), F32),
                 jax.ShapeDtypeStruct((N, H), F32)),
  )(x, W1, degp)


def _tc_mid(p, y, dinv, b, Wn):
  N, H = y.shape

  def body(p_ref, y_ref, dinv_ref, b_ref, w_ref, out_ref):
    acc = p_ref[0] + p_ref[1] + y_ref[...]
    x = acc * dinv_ref[...] + b_ref[...]
    out_ref[...] = jnp.dot(x, w_ref[...], preferred_element_type=F32) * dinv_ref[...]

  return pl.pallas_call(
      body, out_shape=jax.ShapeDtypeStruct((N, H), F32),
  )(p, y, dinv, b, Wn)


def _tc_last(p, y, dinv, b, seg, Wlin, blin):
  N, H = y.shape
  C = Wlin.shape[1]

  def body(p_ref, y_ref, dinv_ref, b_ref, seg_ref, wl_ref, bl_ref, out_ref):
    x3 = (p_ref[0] + p_ref[1] + y_ref[...]) * dinv_ref[...] + b_ref[...]
    gids = lax.broadcasted_iota(jnp.int32, (G, N), 0)
    M = (seg_ref[...] == gids).astype(F32)
    sums = jnp.dot(M, x3, preferred_element_type=F32)
    counts = jnp.sum(M, axis=1, keepdims=True)
    pooled = sums / jnp.maximum(counts, 1.0)
    out_ref[...] = jnp.dot(pooled, wl_ref[...], preferred_element_type=F32) + bl_ref[...]

  return pl.pallas_call(
      body, out_shape=jax.ShapeDtypeStruct((G, C), F32),
  )(p, y, dinv, b, seg, Wlin, blin)


def kernel(inputs, edge_index, batch_indexes, W1, b1, W2, b2, W3, b3, Wlin, blin):
  N, D = inputs.shape
  H = W1.shape[1]
  E = edge_index.shape[1]
  C = Wlin.shape[1]
  NW = NC * NS
  epw = E // NW
  nchunk = epw // CH
  e2 = edge_index.reshape(2, NW, epw)
  em = e2[:, :, :nchunk * CH].reshape(2, NW, nchunk, CH)
  et = e2[:, :, nchunk * CH:]
  srcm, dstm = em[0], em[1]
  srct, dstt = et[0], et[1]

  DW = 8
  _, r_lg = _row_split(N)
  zeros_h = jnp.zeros((r_lg, H), F32)
  zeros_d = jnp.zeros((r_lg, DW), F32)
  ones_d = jnp.ones((CH, DW), F32)

  sc_deg = _make_sc_deg(N, E, DW)
  sc_layer = _make_sc_edge_layer(N, H, E)

  degp = sc_deg(dstm, dstt, ones_d, zeros_d)
  dinv, y1 = _tc_first(inputs, W1, degp)
  p1 = sc_layer(y1, srcm, dstm, srct, dstt, zeros_h)
  y2 = _tc_mid(p1, y1, dinv, b1.reshape(1, H), W2)
  p2 = sc_layer(y2, srcm, dstm, srct, dstt, zeros_h)
  y3 = _tc_mid(p2, y2, dinv, b2.reshape(1, H), W3)
  p3 = sc_layer(y3, srcm, dstm, srct, dstt, zeros_h)
  seg = batch_indexes.reshape(1, N)
  return _tc_last(p3, y3, dinv, b3.reshape(1, H), seg, Wlin, blin.reshape(1, C))
